# Initial kernel scaffold; baseline (speedup 1.0000x reference)
import jax
import jax.numpy as jnp
from jax.experimental import pallas as pl


def kernel(input, table):
    raise NotImplementedError("write your pallas kernel here")


# speedup vs baseline: 4.0857x; 4.0857x over previous
import functools

import jax
import jax.numpy as jnp
from jax import lax
from jax.experimental import pallas as pl
from jax.experimental.pallas import tpu as pltpu
from jax.experimental.pallas import tpu_sc as plsc

NUM_CORES = 2
NUM_SUBCORES = 16
NW = NUM_CORES * NUM_SUBCORES

CHUNK = 128


def _make_kernel(B, D, V):
    assert B % (NW * CHUNK) == 0
    b_per_w = B // NW
    n_chunks = b_per_w // CHUNK
    mesh = plsc.VectorSubcoreMesh(
        core_axis_name="c", subcore_axis_name="s",
        num_cores=NUM_CORES, num_subcores=NUM_SUBCORES)

    @functools.partial(
        pl.kernel,
        out_type=jax.ShapeDtypeStruct((B, D), jnp.float32),
        mesh=mesh,
        compiler_params=pltpu.CompilerParams(use_tc_tiling_on_sc=False),
        scratch_types=[
            pltpu.VMEM((n_chunks, CHUNK), jnp.int32),
            pltpu.VMEM((CHUNK, D), jnp.float32),
            pltpu.SemaphoreType.DMA,
        ],
    )
    def emb(idx_hbm, table_hbm, out_hbm, idx_v, rows_v, gsem):
        wid = lax.axis_index("s") * NUM_CORES + lax.axis_index("c")
        base = wid * b_per_w
        pltpu.sync_copy(idx_hbm.at[wid], idx_v)

        def body(j, carry):
            pltpu.async_copy(table_hbm.at[idx_v.at[j]], rows_v, gsem).wait()
            pltpu.sync_copy(rows_v,
                            out_hbm.at[pl.ds(base + j * CHUNK, CHUNK)])
            return carry

        lax.fori_loop(0, n_chunks, body, 0)

    return emb


def kernel(input, table):
    Bt, H = input.shape
    V, D = table.shape
    B = Bt * H
    idx = input.reshape(NW, (B // NW) // CHUNK, CHUNK)
    out = _make_kernel(B, D, V)(idx, table)
    return out.reshape(Bt, H, D)


# speedup vs baseline: 4.6656x; 1.1419x over previous
import functools

import jax
import jax.numpy as jnp
from jax import lax
from jax.experimental import pallas as pl
from jax.experimental.pallas import tpu as pltpu
from jax.experimental.pallas import tpu_sc as plsc

NUM_CORES = 2
NUM_SUBCORES = 16
NW = NUM_CORES * NUM_SUBCORES

CHUNK = 128


def _make_kernel(B, D, V):
    assert B % (NW * CHUNK) == 0
    b_per_w = B // NW
    n_chunks = b_per_w // CHUNK
    mesh = plsc.VectorSubcoreMesh(
        core_axis_name="c", subcore_axis_name="s",
        num_cores=NUM_CORES, num_subcores=NUM_SUBCORES)

    NBUF = 5
    assert n_chunks % NBUF == 0
    n_groups = n_chunks // NBUF

    @functools.partial(
        pl.kernel,
        out_type=jax.ShapeDtypeStruct((B, D), jnp.float32),
        mesh=mesh,
        compiler_params=pltpu.CompilerParams(use_tc_tiling_on_sc=False),
        scratch_types=[
            pltpu.VMEM((n_chunks, CHUNK), jnp.int32),
            pltpu.VMEM((NBUF, CHUNK, D), jnp.float32),
            pltpu.SemaphoreType.DMA((NBUF,)),
            pltpu.SemaphoreType.DMA((NBUF,)),
        ],
    )
    def emb(idx_hbm, table_hbm, out_hbm, idx_v, rows_v, gsems, ssems):
        wid = lax.axis_index("s") * NUM_CORES + lax.axis_index("c")
        base = wid * b_per_w
        pltpu.sync_copy(idx_hbm.at[wid], idx_v)

        def g(j, b):
            return pltpu.make_async_copy(
                table_hbm.at[idx_v.at[j]], rows_v.at[b], gsems.at[b])

        def s(j, b):
            return pltpu.make_async_copy(
                rows_v.at[b], out_hbm.at[pl.ds(base + j * CHUNK, CHUNK)],
                ssems.at[b])

        for b in range(NBUF):
            g(b, b).start()

        def body(i, carry):
            j0 = i * NBUF
            for b in range(NBUF):
                g(j0 + b, b).wait()
                s(j0 + b, b).start()
            for b in range(NBUF):
                s(j0 + b, b).wait()
                g(j0 + b + NBUF, b).start()
            return carry

        lax.fori_loop(0, n_groups - 1, body, 0)

        j0 = (n_groups - 1) * NBUF
        for b in range(NBUF):
            g(j0 + b, b).wait()
            s(j0 + b, b).start()
        for b in range(NBUF):
            s(j0 + b, b).wait()

    return emb


def kernel(input, table):
    Bt, H = input.shape
    V, D = table.shape
    B = Bt * H
    idx = input.reshape(NW, (B // NW) // CHUNK, CHUNK)
    out = _make_kernel(B, D, V)(idx, table)
    return out.reshape(Bt, H, D)


# speedup vs baseline: 4.7511x; 1.0183x over previous
import functools

import jax
import jax.numpy as jnp
from jax import lax
from jax.experimental import pallas as pl
from jax.experimental.pallas import tpu as pltpu
from jax.experimental.pallas import tpu_sc as plsc

NUM_CORES = 2
NUM_SUBCORES = 16
NW = NUM_CORES * NUM_SUBCORES
LANES = 16

GROUPS_PER_STEP = 16


def _make_kernel(T, D, B, V):
    assert D % NW == 0
    d_per_w = D // NW
    assert B % (LANES * GROUPS_PER_STEP) == 0
    n_steps = B // (LANES * GROUPS_PER_STEP)
    mesh = plsc.VectorSubcoreMesh(
        core_axis_name="c", subcore_axis_name="s",
        num_cores=NUM_CORES, num_subcores=NUM_SUBCORES)

    @functools.partial(
        pl.kernel,
        out_type=jax.ShapeDtypeStruct((T, D, B), jnp.float32),
        mesh=mesh,
        compiler_params=pltpu.CompilerParams(
            use_tc_tiling_on_sc=False, needs_layout_passes=False),
        scratch_types=[
            pltpu.VMEM((V,), jnp.float32),
            pltpu.VMEM((2, B), jnp.int32),
            pltpu.VMEM((2, B), jnp.float32),
            pltpu.SemaphoreType.DMA((2,)),
            pltpu.SemaphoreType.DMA((2,)),
        ],
    )
    def emb(idx_hbm, table_hbm, out_hbm, row_v, idx_v, out_v, isems, osems):
        wid = lax.axis_index("s") * NUM_CORES + lax.axis_index("c")

        def idx_load(t, slot):
            return pltpu.make_async_copy(
                idx_hbm.at[t], idx_v.at[slot], isems.at[slot])

        def out_store(t, d, slot):
            return pltpu.make_async_copy(
                out_v.at[slot], out_hbm.at[t, d], osems.at[slot])

        def gather_step(i, slot):
            base = i * (LANES * GROUPS_PER_STEP)
            for g in range(GROUPS_PER_STEP):
                off = base + g * LANES
                idxs = idx_v[slot, pl.ds(off, LANES)]
                out_v[slot, pl.ds(off, LANES)] = plsc.load_gather(
                    row_v, [idxs])

        for di in range(d_per_w):
            d = wid * d_per_w + di
            pltpu.sync_copy(table_hbm.at[d], row_v)
            idx_load(0, 0).start()

            def body(t, carry, d=d):
                slot = lax.rem(t, 2)
                nxt = lax.rem(t + 1, 2)
                idx_load(t, slot).wait()

                @pl.when(t + 1 < T)
                def _():
                    idx_load(t + 1, nxt).start()

                @pl.when(t >= 2)
                def _():
                    out_store(t - 2, d, slot).wait()

                lax.fori_loop(
                    0, n_steps,
                    lambda i, c: (gather_step(i, slot), c)[1], 0,
                    unroll=True)
                out_store(t, d, slot).start()
                return carry

            lax.fori_loop(0, T, body, 0)
            out_store(T - 2, d, lax.rem(T - 2, 2)).wait()
            out_store(T - 1, d, lax.rem(T - 1, 2)).wait()

    return emb


def kernel(input, table):
    Bt, H = input.shape
    V, D = table.shape
    idxT = input.T
    tableT = table.T
    outT = _make_kernel(H, D, Bt, V)(idxT, tableT)
    return outT.transpose(2, 0, 1)


# speedup vs baseline: 6.1382x; 1.2919x over previous
import functools

import jax
import jax.numpy as jnp
from jax import lax
from jax.experimental import pallas as pl
from jax.experimental.pallas import tpu as pltpu
from jax.experimental.pallas import tpu_sc as plsc

NUM_CORES = 2
NUM_SUBCORES = 16
NW = NUM_CORES * NUM_SUBCORES
LANES = 16

GROUPS_PER_STEP = 16


def _make_kernel(T, D, B, V):
    assert D % NW == 0
    d_per_w = D // NW
    assert B % (LANES * GROUPS_PER_STEP) == 0
    n_steps = B // (LANES * GROUPS_PER_STEP)
    mesh = plsc.VectorSubcoreMesh(
        core_axis_name="c", subcore_axis_name="s",
        num_cores=NUM_CORES, num_subcores=NUM_SUBCORES)

    @functools.partial(
        pl.kernel,
        out_type=jax.ShapeDtypeStruct((T, D, B), jnp.float32),
        mesh=mesh,
        compiler_params=pltpu.CompilerParams(
            use_tc_tiling_on_sc=True, needs_layout_passes=False),
        scratch_types=[
            pltpu.VMEM((V,), jnp.float32),
            pltpu.VMEM((2, B), jnp.int32),
            pltpu.VMEM((2, B), jnp.float32),
            pltpu.SemaphoreType.DMA((2,)),
            pltpu.SemaphoreType.DMA((2,)),
        ],
    )
    def emb(idx_hbm, table_hbm, out_hbm, row_v, idx_v, out_v, isems, osems):
        wid = lax.axis_index("s") * NUM_CORES + lax.axis_index("c")

        def idx_load(t, slot):
            return pltpu.make_async_copy(
                idx_hbm.at[t], idx_v.at[slot], isems.at[slot])

        def out_store(t, d, slot):
            return pltpu.make_async_copy(
                out_v.at[slot], out_hbm.at[t, d], osems.at[slot])

        def gather_step(i, slot):
            base = i * (LANES * GROUPS_PER_STEP)
            for g in range(GROUPS_PER_STEP):
                off = base + g * LANES
                idxs = idx_v[slot, pl.ds(off, LANES)]
                out_v[slot, pl.ds(off, LANES)] = plsc.load_gather(
                    row_v, [idxs])

        for di in range(d_per_w):
            d = wid * d_per_w + di
            pltpu.sync_copy(table_hbm.at[d], row_v)
            idx_load(0, 0).start()

            def body(t, carry, d=d):
                slot = lax.rem(t, 2)
                nxt = lax.rem(t + 1, 2)
                idx_load(t, slot).wait()

                @pl.when(t + 1 < T)
                def _():
                    idx_load(t + 1, nxt).start()

                @pl.when(t >= 2)
                def _():
                    out_store(t - 2, d, slot).wait()

                lax.fori_loop(
                    0, n_steps,
                    lambda i, c: (gather_step(i, slot), c)[1], 0,
                    unroll=True)
                out_store(t, d, slot).start()
                return carry

            lax.fori_loop(0, T, body, 0)
            out_store(T - 2, d, lax.rem(T - 2, 2)).wait()
            out_store(T - 1, d, lax.rem(T - 1, 2)).wait()

    return emb


def kernel(input, table):
    Bt, H = input.shape
    V, D = table.shape
    idxT = input.T
    tableT = table.T
    outT = _make_kernel(H, D, Bt, V)(idxT, tableT)
    return outT.transpose(2, 0, 1)


# speedup vs baseline: 8.1335x; 1.3251x over previous
import functools

import jax
import jax.numpy as jnp
from jax import lax
from jax.experimental import pallas as pl
from jax.experimental.pallas import tpu as pltpu
from jax.experimental.pallas import tpu_sc as plsc

NUM_CORES = 2
NUM_SUBCORES = 16
NW = NUM_CORES * NUM_SUBCORES
LANES = 16

GROUPS_PER_STEP = 16


def _make_kernel(T, D, B, V):
    assert D % NW == 0
    d_per_w = D // NW
    assert B % (LANES * GROUPS_PER_STEP) == 0
    n_steps = B // (LANES * GROUPS_PER_STEP)
    mesh = plsc.VectorSubcoreMesh(
        core_axis_name="c", subcore_axis_name="s",
        num_cores=NUM_CORES, num_subcores=NUM_SUBCORES)

    @functools.partial(
        pl.kernel,
        out_type=jax.ShapeDtypeStruct((T, D, B), jnp.float32),
        mesh=mesh,
        compiler_params=pltpu.CompilerParams(
            use_tc_tiling_on_sc=True, needs_layout_passes=False),
        scratch_types=[
            pltpu.VMEM((V,), jnp.float32),
            pltpu.VMEM((2, B), jnp.int32),
            pltpu.VMEM((2, B), jnp.float32),
            pltpu.SemaphoreType.DMA((2,)),
            pltpu.SemaphoreType.DMA((2,)),
        ],
    )
    def emb(idx_hbm, table_hbm, out_hbm, row_v, idx_v, out_v, isems, osems):
        wid = lax.axis_index("s") * NUM_CORES + lax.axis_index("c")

        def idx_load(t, slot):
            return pltpu.make_async_copy(
                idx_hbm.at[t], idx_v.at[slot], isems.at[slot])

        def out_store(t, d, slot):
            return pltpu.make_async_copy(
                out_v.at[slot], out_hbm.at[t, d], osems.at[slot])

        NG = B // LANES
        PIPE = 4

        def gather_row(slot):
            idx_pend = {}
            val_pend = {}
            for g in range(PIPE):
                idx_pend[g] = idx_v[slot, pl.ds(g * LANES, LANES)]
            for g in range(NG + 1):
                if g + PIPE < NG:
                    idx_pend[g + PIPE] = idx_v[
                        slot, pl.ds((g + PIPE) * LANES, LANES)]
                if g < NG:
                    val_pend[g] = plsc.load_gather(row_v, [idx_pend.pop(g)])
                if g >= 1:
                    out_v[slot, pl.ds((g - 1) * LANES, LANES)] = (
                        val_pend.pop(g - 1))

        for di in range(d_per_w):
            d = wid * d_per_w + di
            pltpu.sync_copy(table_hbm.at[d], row_v)
            idx_load(0, 0).start()

            def body(t, carry, d=d):
                slot = lax.rem(t, 2)
                nxt = lax.rem(t + 1, 2)
                idx_load(t, slot).wait()

                @pl.when(t + 1 < T)
                def _():
                    idx_load(t + 1, nxt).start()

                @pl.when(t >= 2)
                def _():
                    out_store(t - 2, d, slot).wait()

                gather_row(slot)
                out_store(t, d, slot).start()
                return carry

            lax.fori_loop(0, T, body, 0)
            out_store(T - 2, d, lax.rem(T - 2, 2)).wait()
            out_store(T - 1, d, lax.rem(T - 1, 2)).wait()

    return emb


def kernel(input, table):
    Bt, H = input.shape
    V, D = table.shape
    idxT = input.T
    tableT = table.T
    outT = _make_kernel(H, D, Bt, V)(idxT, tableT)
    return outT.transpose(2, 0, 1)


# speedup vs baseline: 8.2921x; 1.0195x over previous
import functools

import jax
import jax.numpy as jnp
from jax import lax
from jax.experimental import pallas as pl
from jax.experimental.pallas import tpu as pltpu
from jax.experimental.pallas import tpu_sc as plsc

NUM_CORES = 2
NUM_SUBCORES = 16
NW = NUM_CORES * NUM_SUBCORES
LANES = 16

GROUPS_PER_STEP = 16


def _make_kernel(T, D, B, V):
    assert D % NW == 0
    d_per_w = D // NW
    assert B % (LANES * GROUPS_PER_STEP) == 0
    n_steps = B // (LANES * GROUPS_PER_STEP)
    mesh = plsc.VectorSubcoreMesh(
        core_axis_name="c", subcore_axis_name="s",
        num_cores=NUM_CORES, num_subcores=NUM_SUBCORES)

    @functools.partial(
        pl.kernel,
        out_type=jax.ShapeDtypeStruct((T, D, B), jnp.float32),
        mesh=mesh,
        compiler_params=pltpu.CompilerParams(
            use_tc_tiling_on_sc=True, needs_layout_passes=False),
        scratch_types=[
            pltpu.VMEM((V,), jnp.float32),
            pltpu.VMEM((2, B), jnp.int32),
            pltpu.VMEM((2, B), jnp.float32),
            pltpu.SemaphoreType.DMA((2,)),
            pltpu.SemaphoreType.DMA((2,)),
        ],
    )
    def emb(idx_hbm, table_hbm, out_hbm, row_v, idx_v, out_v, isems, osems):
        wid = lax.axis_index("s") * NUM_CORES + lax.axis_index("c")

        def idx_load(t, slot):
            return pltpu.make_async_copy(
                idx_hbm.at[t], idx_v.at[slot], isems.at[slot])

        def out_store(t, d, slot):
            return pltpu.make_async_copy(
                out_v.at[slot], out_hbm.at[t, d], osems.at[slot])

        NG = B // LANES
        PIPE = 8
        SLAG = 2

        def gather_row(slot):
            idx_pend = {}
            val_pend = {}
            for g in range(PIPE):
                idx_pend[g] = idx_v[slot, pl.ds(g * LANES, LANES)]
            for g in range(NG + SLAG):
                if g + PIPE < NG:
                    idx_pend[g + PIPE] = idx_v[
                        slot, pl.ds((g + PIPE) * LANES, LANES)]
                if g < NG:
                    val_pend[g] = plsc.load_gather(row_v, [idx_pend.pop(g)])
                if g >= SLAG:
                    out_v[slot, pl.ds((g - SLAG) * LANES, LANES)] = (
                        val_pend.pop(g - SLAG))

        for di in range(d_per_w):
            d = wid * d_per_w + di
            pltpu.sync_copy(table_hbm.at[d], row_v)
            idx_load(0, 0).start()

            def body(t, carry, d=d):
                slot = lax.rem(t, 2)
                nxt = lax.rem(t + 1, 2)
                idx_load(t, slot).wait()

                @pl.when(t + 1 < T)
                def _():
                    idx_load(t + 1, nxt).start()

                @pl.when(t >= 2)
                def _():
                    out_store(t - 2, d, slot).wait()

                gather_row(slot)
                out_store(t, d, slot).start()
                return carry

            lax.fori_loop(0, T, body, 0)
            out_store(T - 2, d, lax.rem(T - 2, 2)).wait()
            out_store(T - 1, d, lax.rem(T - 1, 2)).wait()

    return emb


def kernel(input, table):
    Bt, H = input.shape
    V, D = table.shape
    idxT = input.T
    tableT = table.T
    outT = _make_kernel(H, D, Bt, V)(idxT, tableT)
    return outT.transpose(2, 0, 1)


# speedup vs baseline: 12.1909x; 1.4702x over previous
import functools

import jax
import jax.numpy as jnp
from jax import lax
from jax.experimental import pallas as pl
from jax.experimental.pallas import tpu as pltpu
from jax.experimental.pallas import tpu_sc as plsc

NUM_CORES = 2
NUM_SUBCORES = 16
NW = NUM_CORES * NUM_SUBCORES
LANES = 16

GROUPS_PER_STEP = 16


def _make_kernel(T, D, B, V):
    assert D % NW == 0
    d_per_w = D // NW
    assert B % (LANES * GROUPS_PER_STEP) == 0
    n_steps = B // (LANES * GROUPS_PER_STEP)
    mesh = plsc.VectorSubcoreMesh(
        core_axis_name="c", subcore_axis_name="s",
        num_cores=NUM_CORES, num_subcores=NUM_SUBCORES)

    @functools.partial(
        pl.kernel,
        out_type=jax.ShapeDtypeStruct((T, D, B), jnp.float32),
        mesh=mesh,
        compiler_params=pltpu.CompilerParams(
            use_tc_tiling_on_sc=True, needs_layout_passes=False),
        scratch_types=[
            pltpu.VMEM((V,), jnp.float32),
            pltpu.VMEM((2, B), jnp.int32),
            pltpu.VMEM((2, B), jnp.float32),
            pltpu.VMEM_SHARED((T, B), jnp.int32),
            pltpu.SemaphoreType.DMA((2,)),
            pltpu.SemaphoreType.DMA((2,)),
        ],
    )
    def emb(idx_hbm, table_hbm, out_hbm, row_v, idx_v, out_v, idx_sh,
            isems, osems):
        sid = lax.axis_index("s")
        wid = sid * NUM_CORES + lax.axis_index("c")

        n_full = (T // 8) * 8

        @pl.when(sid < 6)
        def _():
            r0 = sid * 8
            pltpu.sync_copy(idx_hbm.at[pl.ds(r0, 8)],
                            idx_sh.at[pl.ds(r0, 8)])

        @pl.when(sid == 6)
        def _():
            pltpu.sync_copy(idx_hbm.at[pl.ds(n_full, T - n_full)],
                            idx_sh.at[pl.ds(n_full, T - n_full)])

        plsc.subcore_barrier()

        def idx_load(t, slot):
            return pltpu.make_async_copy(
                idx_sh.at[t], idx_v.at[slot], isems.at[slot])

        def out_store(t, d, slot):
            return pltpu.make_async_copy(
                out_v.at[slot], out_hbm.at[t, d], osems.at[slot])

        NG = B // LANES
        PIPE = 8
        SLAG = 2

        def gather_row(slot):
            idx_pend = {}
            val_pend = {}
            for g in range(PIPE):
                idx_pend[g] = idx_v[slot, pl.ds(g * LANES, LANES)]
            for g in range(NG + SLAG):
                if g + PIPE < NG:
                    idx_pend[g + PIPE] = idx_v[
                        slot, pl.ds((g + PIPE) * LANES, LANES)]
                if g < NG:
                    val_pend[g] = plsc.load_gather(row_v, [idx_pend.pop(g)])
                if g >= SLAG:
                    out_v[slot, pl.ds((g - SLAG) * LANES, LANES)] = (
                        val_pend.pop(g - SLAG))

        for di in range(d_per_w):
            d = wid * d_per_w + di
            pltpu.sync_copy(table_hbm.at[d], row_v)
            idx_load(0, 0).start()

            def body(t, carry, d=d):
                slot = lax.rem(t, 2)
                nxt = lax.rem(t + 1, 2)
                idx_load(t, slot).wait()

                @pl.when(t + 1 < T)
                def _():
                    idx_load(t + 1, nxt).start()

                @pl.when(t >= 2)
                def _():
                    out_store(t - 2, d, slot).wait()

                gather_row(slot)
                out_store(t, d, slot).start()
                return carry

            lax.fori_loop(0, T, body, 0)
            out_store(T - 2, d, lax.rem(T - 2, 2)).wait()
            out_store(T - 1, d, lax.rem(T - 1, 2)).wait()

    return emb


def kernel(input, table):
    Bt, H = input.shape
    V, D = table.shape
    idxT = input.T
    tableT = table.T
    outT = _make_kernel(H, D, Bt, V)(idxT, tableT)
    return outT.transpose(2, 0, 1)


# speedup vs baseline: 12.1928x; 1.0002x over previous
import functools

import jax
import jax.numpy as jnp
from jax import lax
from jax.experimental import pallas as pl
from jax.experimental.pallas import tpu as pltpu
from jax.experimental.pallas import tpu_sc as plsc

NUM_CORES = 2
NUM_SUBCORES = 16
NW = NUM_CORES * NUM_SUBCORES
LANES = 16

GROUPS_PER_STEP = 16


def _make_kernel(T, D, B, V):
    assert D % NW == 0
    d_per_w = D // NW
    assert B % (LANES * GROUPS_PER_STEP) == 0
    n_steps = B // (LANES * GROUPS_PER_STEP)
    mesh = plsc.VectorSubcoreMesh(
        core_axis_name="c", subcore_axis_name="s",
        num_cores=NUM_CORES, num_subcores=NUM_SUBCORES)

    @functools.partial(
        pl.kernel,
        out_type=jax.ShapeDtypeStruct((T, D, B), jnp.float32),
        mesh=mesh,
        compiler_params=pltpu.CompilerParams(
            use_tc_tiling_on_sc=True, needs_layout_passes=False),
        scratch_types=[
            pltpu.VMEM((V,), jnp.float32),
            pltpu.VMEM((2, B), jnp.int32),
            pltpu.VMEM((2, B), jnp.float32),
            pltpu.VMEM_SHARED((((T + 7) // 8) * 8, B), jnp.int32),
            pltpu.SemaphoreType.DMA((2,)),
            pltpu.SemaphoreType.DMA((2,)),
        ],
    )
    def emb(idx_hbm, table_hbm, out_hbm, row_v, idx_v, out_v, idx_sh,
            isems, osems):
        sid = lax.axis_index("s")
        wid = sid * NUM_CORES + lax.axis_index("c")

        n_full = (T // 8) * 8

        @pl.when(sid < 6)
        def _():
            r0 = sid * 8
            pltpu.sync_copy(idx_hbm.at[pl.ds(r0, 8)],
                            idx_sh.at[pl.ds(r0, 8)])

        @pl.when(sid == 6)
        def _():
            pltpu.sync_copy(idx_hbm.at[pl.ds(n_full, T - n_full)],
                            idx_sh.at[pl.ds(n_full, T - n_full)])

        plsc.subcore_barrier()

        def idx_load(t, slot):
            return pltpu.make_async_copy(
                idx_sh.at[t], idx_v.at[slot], isems.at[slot])

        def out_store(t, d, slot):
            return pltpu.make_async_copy(
                out_v.at[slot], out_hbm.at[t, d], osems.at[slot])

        NG = B // LANES
        PIPE = 8
        SLAG = 2

        def gather_row(slot):
            idx_pend = {}
            val_pend = {}
            for g in range(PIPE):
                idx_pend[g] = idx_v[slot, pl.ds(g * LANES, LANES)]
            for g in range(NG + SLAG):
                if g + PIPE < NG:
                    idx_pend[g + PIPE] = idx_v[
                        slot, pl.ds((g + PIPE) * LANES, LANES)]
                if g < NG:
                    val_pend[g] = plsc.load_gather(row_v, [idx_pend.pop(g)])
                if g >= SLAG:
                    out_v[slot, pl.ds((g - SLAG) * LANES, LANES)] = (
                        val_pend.pop(g - SLAG))

        for di in range(d_per_w):
            d = wid * d_per_w + di
            pltpu.sync_copy(table_hbm.at[d], row_v)
            idx_load(0, 0).start()

            def body(t, carry, d=d):
                slot = lax.rem(t, 2)
                nxt = lax.rem(t + 1, 2)
                idx_load(t, slot).wait()

                @pl.when(t + 1 < T)
                def _():
                    idx_load(t + 1, nxt).start()

                @pl.when(t >= 2)
                def _():
                    out_store(t - 2, d, slot).wait()

                gather_row(slot)
                out_store(t, d, slot).start()
                return carry

            lax.fori_loop(0, T, body, 0)
            out_store(T - 2, d, lax.rem(T - 2, 2)).wait()
            out_store(T - 1, d, lax.rem(T - 1, 2)).wait()

    return emb


def kernel(input, table):
    Bt, H = input.shape
    V, D = table.shape
    idxT = input.T
    tableT = table.T
    outT = _make_kernel(H, D, Bt, V)(idxT, tableT)
    return outT.transpose(2, 0, 1)


# speedup vs baseline: 12.3840x; 1.0157x over previous
import functools

import jax
import jax.numpy as jnp
from jax import lax
from jax.experimental import pallas as pl
from jax.experimental.pallas import tpu as pltpu
from jax.experimental.pallas import tpu_sc as plsc

NUM_CORES = 2
NUM_SUBCORES = 16
NW = NUM_CORES * NUM_SUBCORES
LANES = 16

GROUPS_PER_STEP = 16


def _make_kernel(T, D, B, V):
    assert D % NW == 0
    d_per_w = D // NW
    assert B % (LANES * GROUPS_PER_STEP) == 0
    n_steps = B // (LANES * GROUPS_PER_STEP)
    mesh = plsc.VectorSubcoreMesh(
        core_axis_name="c", subcore_axis_name="s",
        num_cores=NUM_CORES, num_subcores=NUM_SUBCORES)

    @functools.partial(
        pl.kernel,
        out_type=jax.ShapeDtypeStruct((T, D, B), jnp.float32),
        mesh=mesh,
        compiler_params=pltpu.CompilerParams(
            use_tc_tiling_on_sc=True, needs_layout_passes=False),
        scratch_types=[
            pltpu.VMEM((V,), jnp.float32),
            pltpu.VMEM((2, B), jnp.int32),
            pltpu.VMEM((2, B), jnp.float32),
            pltpu.VMEM_SHARED((((T + 7) // 8) * 8, B), jnp.int32),
            pltpu.SemaphoreType.DMA((2,)),
            pltpu.SemaphoreType.DMA((2,)),
            pltpu.SemaphoreType.DMA,
        ],
    )
    def emb(idx_hbm, table_hbm, out_hbm, row_v, idx_v, out_v, idx_sh,
            isems, osems, rsem):
        sid = lax.axis_index("s")
        wid = sid * NUM_CORES + lax.axis_index("c")

        first_row = pltpu.make_async_copy(
            table_hbm.at[wid * d_per_w], row_v, rsem)
        first_row.start()

        n_full = (T // 8) * 8

        @pl.when(sid < 6)
        def _():
            r0 = sid * 8
            pltpu.sync_copy(idx_hbm.at[pl.ds(r0, 8)],
                            idx_sh.at[pl.ds(r0, 8)])

        @pl.when(sid == 6)
        def _():
            pltpu.sync_copy(idx_hbm.at[pl.ds(n_full, T - n_full)],
                            idx_sh.at[pl.ds(n_full, T - n_full)])

        plsc.subcore_barrier()

        def idx_load(t, slot):
            return pltpu.make_async_copy(
                idx_sh.at[t], idx_v.at[slot], isems.at[slot])

        def out_store(t, d, slot):
            return pltpu.make_async_copy(
                out_v.at[slot], out_hbm.at[t, d], osems.at[slot])

        NG = B // LANES
        PIPE = 8
        SLAG = 2

        def gather_row(slot):
            idx_pend = {}
            val_pend = {}
            for g in range(PIPE):
                idx_pend[g] = idx_v[slot, pl.ds(g * LANES, LANES)]
            for g in range(NG + SLAG):
                if g + PIPE < NG:
                    idx_pend[g + PIPE] = idx_v[
                        slot, pl.ds((g + PIPE) * LANES, LANES)]
                if g < NG:
                    val_pend[g] = plsc.load_gather(row_v, [idx_pend.pop(g)])
                if g >= SLAG:
                    out_v[slot, pl.ds((g - SLAG) * LANES, LANES)] = (
                        val_pend.pop(g - SLAG))

        for di in range(d_per_w):
            d = wid * d_per_w + di
            if di == 0:
                first_row.wait()
            else:
                pltpu.sync_copy(table_hbm.at[d], row_v)
            idx_load(0, 0).start()

            def body(t, carry, d=d):
                slot = lax.rem(t, 2)
                nxt = lax.rem(t + 1, 2)
                idx_load(t, slot).wait()

                @pl.when(t + 1 < T)
                def _():
                    idx_load(t + 1, nxt).start()

                @pl.when(t >= 2)
                def _():
                    out_store(t - 2, d, slot).wait()

                gather_row(slot)
                out_store(t, d, slot).start()
                return carry

            lax.fori_loop(0, T, body, 0)
            out_store(T - 2, d, lax.rem(T - 2, 2)).wait()
            out_store(T - 1, d, lax.rem(T - 1, 2)).wait()

    return emb


def kernel(input, table):
    Bt, H = input.shape
    V, D = table.shape
    idxT = input.T
    tableT = table.T
    outT = _make_kernel(H, D, Bt, V)(idxT, tableT)
    return outT.transpose(2, 0, 1)
